# baseline (speedup 1.0000x reference)
import functools

import jax
import jax.numpy as jnp
from jax import lax
from jax.experimental import pallas as pl
from jax.experimental.pallas import tpu as pltpu
from jax.experimental.pallas import tpu_sc as plsc

N = 10000
E = 320000
CPB = 4
NP = 2
CHUNK = 4000
NCH = E // CHUNK
NV = CHUNK // 16
EPW = E // 32
CHUNK_C = 2000
NCH_C = EPW // CHUNK_C

_mesh = plsc.VectorSubcoreMesh(core_axis_name="c", subcore_axis_name="s")


def _count_body(dst_hbm, zeros_hbm, out_hbm, dbuf, acc, sem):
    c = lax.axis_index("c")
    s = lax.axis_index("s")
    wid = c * 16 + s
    pltpu.sync_copy(zeros_hbm, acc)
    row0 = jnp.zeros((16,), jnp.int32)
    one = jnp.ones((16,), jnp.float32)

    def chunk_body(g, carry):
        pltpu.sync_copy(dst_hbm.at[pl.ds(wid * EPW + g * CHUNK_C, CHUNK_C)], dbuf)

        @plsc.parallel_loop(0, CHUNK_C // 16, unroll=8)
        def vec_body(j):
            d16 = dbuf[pl.ds(j * 16, 16)]
            plsc.addupdate_scatter(acc, [row0, d16], one)

        return carry

    lax.fori_loop(0, NCH_C, chunk_body, 0)
    pltpu.sync_copy(acc, out_hbm.at[wid])


@functools.partial(
    pl.kernel,
    out_type=jax.ShapeDtypeStruct((32, 1, N), jnp.float32),
    mesh=_mesh,
    compiler_params=pltpu.CompilerParams(needs_layout_passes=False),
    scratch_types=[
        pltpu.VMEM((CHUNK_C,), jnp.int32),
        pltpu.VMEM((1, N), jnp.float32),
        pltpu.SemaphoreType.DMA,
    ],
)
def _count_kernel(dst_hbm, zeros_hbm, out_hbm, dbuf, acc, sem):
    _count_body(dst_hbm, zeros_hbm, out_hbm, dbuf, acc, sem)


def _aggc_body(hc_hbm, src_hbm, dst_hbm, zeros_hbm, out_hbm,
               sbuf0, sbuf1, dbuf0, dbuf1, rows, acc,
               sem, ss0, ss1, sd0, sd1):
    c = lax.axis_index("c")
    s = lax.axis_index("s")
    wid = c * 16 + s
    sbufs = (sbuf0, sbuf1)
    dbufs = (dbuf0, dbuf1)
    ssems = (ss0, ss1)
    dsems = (sd0, sd1)

    def start(g, b):
        pltpu.async_copy(src_hbm.at[pl.ds(g * CHUNK, CHUNK)], sbufs[b], ssems[b])
        pltpu.async_copy(dst_hbm.at[pl.ds(g * CHUNK, CHUNK)], dbufs[b], dsems[b])

    def wait(b):
        pltpu.make_async_copy(src_hbm.at[pl.ds(0, CHUNK)], sbufs[b], ssems[b]).wait()
        pltpu.make_async_copy(dst_hbm.at[pl.ds(0, CHUNK)], dbufs[b], dsems[b]).wait()

    def compute(b):
        sb = sbufs[b]
        db = dbufs[b]

        @plsc.parallel_loop(0, NV, unroll=8)
        def vec_body(j):
            s16 = sb[pl.ds(j * 16, 16)]
            d16 = db[pl.ds(j * 16, 16)]
            for col in range(CPB):
                cvec = jnp.full((16,), col, jnp.int32)
                v = plsc.load_gather(rows, [cvec, s16])
                plsc.addupdate_scatter(acc, [cvec, d16], v)

    for p in range(NP):
        blk = p * 32 + wid
        pltpu.sync_copy(hc_hbm.at[blk], rows)
        pltpu.sync_copy(zeros_hbm, acc)
        start(0, 0)

        def chunk_body(t, carry):
            g0 = 2 * t
            start(g0 + 1, 1)
            wait(0)
            compute(0)

            @pl.when(g0 + 2 < NCH)
            def _():
                start(g0 + 2, 0)

            wait(1)
            compute(1)
            return carry

        lax.fori_loop(0, NCH // 2, chunk_body, 0)
        pltpu.sync_copy(acc, out_hbm.at[blk])


@functools.partial(
    pl.kernel,
    out_type=jax.ShapeDtypeStruct((64, CPB, N), jnp.float32),
    mesh=_mesh,
    compiler_params=pltpu.CompilerParams(needs_layout_passes=False),
    scratch_types=[
        pltpu.VMEM((CHUNK,), jnp.int32),
        pltpu.VMEM((CHUNK,), jnp.int32),
        pltpu.VMEM((CHUNK,), jnp.int32),
        pltpu.VMEM((CHUNK,), jnp.int32),
        pltpu.VMEM((CPB, N), jnp.float32),
        pltpu.VMEM((CPB, N), jnp.float32),
        pltpu.SemaphoreType.DMA,
        pltpu.SemaphoreType.DMA,
        pltpu.SemaphoreType.DMA,
        pltpu.SemaphoreType.DMA,
        pltpu.SemaphoreType.DMA,
    ],
)
def _aggc_kernel(hc_hbm, src_hbm, dst_hbm, zeros_hbm, out_hbm,
                 sbuf0, sbuf1, dbuf0, dbuf1, rows, acc,
                 sem, ss0, ss1, sd0, sd1):
    _aggc_body(hc_hbm, src_hbm, dst_hbm, zeros_hbm, out_hbm,
               sbuf0, sbuf1, dbuf0, dbuf1, rows, acc,
               sem, ss0, ss1, sd0, sd1)


def _enc_body(x_ref, w_ref, b_ref, g_ref, beta_ref, h_ref):
    h = jnp.dot(x_ref[...], w_ref[...].T, preferred_element_type=jnp.float32)
    h = h + b_ref[...][None, :]
    mu = jnp.mean(h, axis=0, keepdims=True)
    var = jnp.mean((h - mu) ** 2, axis=0, keepdims=True)
    h_ref[...] = (g_ref[...][None, :] * (h - mu) / jnp.sqrt(var + 1e-5)
                  + beta_ref[...][None, :])


def _encoder(x, w, b, g, beta):
    return pl.pallas_call(
        _enc_body,
        out_shape=jax.ShapeDtypeStruct((N, 256), jnp.float32),
    )(x, w, b, g, beta)


def _tr_body(x_ref, out_ref):
    out_ref[...] = x_ref[...].T


def _transpose(h):
    return pl.pallas_call(
        _tr_body,
        out_shape=jax.ShapeDtypeStruct((256, N), jnp.float32),
    )(h)


def _prep_body(cnt_ref, inv_ref):
    cnt = jnp.sum(cnt_ref[...], axis=0, keepdims=True)
    inv_ref[...] = 1.0 / jnp.maximum(cnt, 1.0)


def _prep(cnt_parts):
    return pl.pallas_call(
        _prep_body,
        out_shape=jax.ShapeDtypeStruct((1, N), jnp.float32),
    )(cnt_parts)


def _layer_body(aggc_ref, inv_ref, h_ref, wl_ref, bl_ref, wr_ref, out_ref):
    aggs = aggc_ref[...] * inv_ref[...]
    z = lax.dot_general(aggs, wl_ref[...], (((0,), (1,)), ((), ())),
                        preferred_element_type=jnp.float32)
    out = (z + bl_ref[...][None, :]
           + jnp.dot(h_ref[...], wr_ref[...].T,
                     preferred_element_type=jnp.float32))
    out_ref[...] = jnp.maximum(out, 0.0)


def _layer(aggc, inv, h, wl, bl, wr):
    return pl.pallas_call(
        _layer_body,
        out_shape=jax.ShapeDtypeStruct((N, 256), jnp.float32),
    )(aggc, inv, h, wl, bl, wr)


def _head_body(x1_ref, x2_ref, x3_ref, wjk_ref, bjk_ref,
               wc0, bc0, gc0, bec0, wc1, bc1, gc1, bec1,
               wc2, bc2, gc2, bec2, wc3, bc3, gc3, bec3,
               wo_ref, bo_ref, out_ref):
    wjk = wjk_ref[...]
    h = (jnp.dot(x1_ref[...], wjk[:, 0:256].T, preferred_element_type=jnp.float32)
         + jnp.dot(x2_ref[...], wjk[:, 256:512].T, preferred_element_type=jnp.float32)
         + jnp.dot(x3_ref[...], wjk[:, 512:768].T, preferred_element_type=jnp.float32)
         + bjk_ref[...][None, :])
    for (wc, bc, gc, bec) in ((wc0, bc0, gc0, bec0), (wc1, bc1, gc1, bec1),
                              (wc2, bc2, gc2, bec2), (wc3, bc3, gc3, bec3)):
        h = jnp.dot(h, wc[...].T, preferred_element_type=jnp.float32) + bc[...][None, :]
        mu = jnp.mean(h, axis=0, keepdims=True)
        var = jnp.mean((h - mu) ** 2, axis=0, keepdims=True)
        h = gc[...][None, :] * (h - mu) / jnp.sqrt(var + 1e-5) + bec[...][None, :]
        h = jnp.maximum(h, 0.0)
    o = jnp.sum(h * wo_ref[...], axis=1, keepdims=True) + bo_ref[...]
    out_ref[...] = jax.nn.sigmoid(o)


def _head(x1, x2, x3, wjk, bjk, clf, wo, bo):
    flat = []
    for t in clf:
        flat.extend(t)
    return pl.pallas_call(
        _head_body,
        out_shape=jax.ShapeDtypeStruct((N, 1), jnp.float32),
    )(x1, x2, x3, wjk, bjk, *flat, wo, bo)


def kernel(x, edge_index, W_enc, b_enc, g_enc, beta_enc,
           Wl0, bl0, Wr0, Wl1, bl1, Wr1, Wl2, bl2, Wr2,
           W_jk, b_jk,
           Wc0, bc0, gc0, betac0, Wc1, bc1, gc1, betac1,
           Wc2, bc2, gc2, betac2, Wc3, bc3, gc3, betac3,
           W_out, b_out):
    ei = edge_index.astype(jnp.int32)
    src = ei[0]
    dst = ei[1]
    h = _encoder(x, W_enc, b_enc, g_enc, beta_enc)

    zeros1 = jnp.zeros((1, N), jnp.float32)
    zeros4 = jnp.zeros((CPB, N), jnp.float32)

    cnt_parts = _count_kernel(dst, zeros1)
    inv = _prep(cnt_parts.reshape(32, N))

    xs = []
    for (wl, bl, wr) in ((Wl0, bl0, Wr0), (Wl1, bl1, Wr1), (Wl2, bl2, Wr2)):
        hc = _transpose(h).reshape(64, CPB, N)
        aggc = _aggc_kernel(hc, src, dst, zeros4).reshape(256, N)
        h = _layer(aggc, inv, h, wl, bl, wr)
        xs.append(h)

    clf = ((Wc0, bc0, gc0, betac0), (Wc1, bc1, gc1, betac1),
           (Wc2, bc2, gc2, betac2), (Wc3, bc3, gc3, betac3))
    return _head(xs[0], xs[1], xs[2], W_jk, b_jk, clf, W_out,
                 b_out.reshape(1, 1))


# speedup vs baseline: 1.0482x; 1.0482x over previous
import functools

import jax
import jax.numpy as jnp
from jax import lax
from jax.experimental import pallas as pl
from jax.experimental.pallas import tpu as pltpu
from jax.experimental.pallas import tpu_sc as plsc

N = 10000
E = 320000
CPB = 4
NP = 2
CHUNK = 4000
NCH = E // CHUNK
NV = CHUNK // 16
EPW = E // 32
EPW2 = E // 16
CHUNK_C = 2000
NCH_C = EPW // CHUNK_C

_mesh = plsc.VectorSubcoreMesh(core_axis_name="c", subcore_axis_name="s")


def _count_body(dst_hbm, zeros_hbm, out_hbm, dbuf, acc, sem):
    c = lax.axis_index("c")
    s = lax.axis_index("s")
    wid = c * 16 + s
    pltpu.sync_copy(zeros_hbm, acc)
    row0 = jnp.zeros((16,), jnp.int32)
    one = jnp.ones((16,), jnp.float32)

    def chunk_body(g, carry):
        pltpu.sync_copy(dst_hbm.at[pl.ds(wid * EPW + g * CHUNK_C, CHUNK_C)], dbuf)

        @plsc.parallel_loop(0, CHUNK_C // 16, unroll=8)
        def vec_body(j):
            d16 = dbuf[pl.ds(j * 16, 16)]
            plsc.addupdate_scatter(acc, [row0, d16], one)

        return carry

    lax.fori_loop(0, NCH_C, chunk_body, 0)
    pltpu.sync_copy(acc, out_hbm.at[wid])


@functools.partial(
    pl.kernel,
    out_type=jax.ShapeDtypeStruct((32, 1, N), jnp.float32),
    mesh=_mesh,
    compiler_params=pltpu.CompilerParams(needs_layout_passes=False),
    scratch_types=[
        pltpu.VMEM((CHUNK_C,), jnp.int32),
        pltpu.VMEM((1, N), jnp.float32),
        pltpu.SemaphoreType.DMA,
    ],
)
def _count_kernel(dst_hbm, zeros_hbm, out_hbm, dbuf, acc, sem):
    _count_body(dst_hbm, zeros_hbm, out_hbm, dbuf, acc, sem)


def _aggc_body(hc_hbm, pk_hbm, zeros_hbm, out_hbm,
               pbuf0, pbuf1, rows, acc, pk_sp,
               sem, ss0, ss1):
    c = lax.axis_index("c")
    s = lax.axis_index("s")
    wid = c * 16 + s

    for q in range(EPW2 // CHUNK):
        off = s * EPW2 + q * CHUNK
        pltpu.sync_copy(pk_hbm.at[pl.ds(off, CHUNK)], pbuf0)
        pltpu.sync_copy(pbuf0, pk_sp.at[pl.ds(off, CHUNK)])
    plsc.subcore_barrier()

    pbufs = (pbuf0, pbuf1)
    ssems = (ss0, ss1)

    def start(g, b):
        pltpu.async_copy(pk_sp.at[pl.ds(g * CHUNK, CHUNK)], pbufs[b], ssems[b])

    def wait(b):
        pltpu.make_async_copy(pk_sp.at[pl.ds(0, CHUNK)], pbufs[b], ssems[b]).wait()

    def compute(b):
        pb = pbufs[b]

        @plsc.parallel_loop(0, NV, unroll=4)
        def vec_body(j):
            p16 = pb[pl.ds(j * 16, 16)]
            s16 = jax.lax.shift_right_logical(p16, 14)
            d16 = p16 & 16383
            for col in range(CPB):
                cvec = jnp.full((16,), col, jnp.int32)
                v = plsc.load_gather(rows, [cvec, s16])
                plsc.addupdate_scatter(acc, [cvec, d16], v)

    for p in range(NP):
        blk = p * 32 + wid
        pltpu.sync_copy(hc_hbm.at[blk], rows)
        pltpu.sync_copy(zeros_hbm, acc)
        start(0, 0)

        def chunk_body(t, carry):
            g0 = 2 * t
            start(g0 + 1, 1)
            wait(0)
            compute(0)

            @pl.when(g0 + 2 < NCH)
            def _():
                start(g0 + 2, 0)

            wait(1)
            compute(1)
            return carry

        lax.fori_loop(0, NCH // 2, chunk_body, 0)
        pltpu.sync_copy(acc, out_hbm.at[blk])


@functools.partial(
    pl.kernel,
    out_type=jax.ShapeDtypeStruct((64, CPB, N), jnp.float32),
    mesh=_mesh,
    compiler_params=pltpu.CompilerParams(needs_layout_passes=False),
    scratch_types=[
        pltpu.VMEM((CHUNK,), jnp.int32),
        pltpu.VMEM((CHUNK,), jnp.int32),
        pltpu.VMEM((CPB, N), jnp.float32),
        pltpu.VMEM((CPB, N), jnp.float32),
        pltpu.VMEM_SHARED((E,), jnp.int32),
        pltpu.SemaphoreType.DMA,
        pltpu.SemaphoreType.DMA,
        pltpu.SemaphoreType.DMA,
    ],
)
def _aggc_kernel(hc_hbm, pk_hbm, zeros_hbm, out_hbm,
                 pbuf0, pbuf1, rows, acc, pk_sp,
                 sem, ss0, ss1):
    _aggc_body(hc_hbm, pk_hbm, zeros_hbm, out_hbm,
               pbuf0, pbuf1, rows, acc, pk_sp,
               sem, ss0, ss1)


def _enc_body(x_ref, w_ref, b_ref, g_ref, beta_ref, ei_ref, h_ref, pk_ref):
    h = jnp.dot(x_ref[...], w_ref[...].T, preferred_element_type=jnp.float32)
    h = h + b_ref[...][None, :]
    mu = jnp.mean(h, axis=0, keepdims=True)
    var = jnp.mean((h - mu) ** 2, axis=0, keepdims=True)
    h_ref[...] = (g_ref[...][None, :] * (h - mu) / jnp.sqrt(var + 1e-5)
                  + beta_ref[...][None, :])
    pk_ref[...] = ei_ref[0:1, :] * 16384 + ei_ref[1:2, :]


def _encoder(x, w, b, g, beta, ei):
    return pl.pallas_call(
        _enc_body,
        out_shape=[
            jax.ShapeDtypeStruct((N, 256), jnp.float32),
            jax.ShapeDtypeStruct((1, E), jnp.int32),
        ],
    )(x, w, b, g, beta, ei)


def _tr_body(x_ref, out_ref):
    out_ref[...] = x_ref[...].T


def _transpose(h):
    return pl.pallas_call(
        _tr_body,
        out_shape=jax.ShapeDtypeStruct((256, N), jnp.float32),
    )(h)


def _prep_body(cnt_ref, inv_ref):
    cnt = jnp.sum(cnt_ref[...], axis=0, keepdims=True)
    inv_ref[...] = 1.0 / jnp.maximum(cnt, 1.0)


def _prep(cnt_parts):
    return pl.pallas_call(
        _prep_body,
        out_shape=jax.ShapeDtypeStruct((1, N), jnp.float32),
    )(cnt_parts)


def _layer_body(aggc_ref, inv_ref, h_ref, wl_ref, bl_ref, wr_ref, out_ref):
    aggs = aggc_ref[...] * inv_ref[...]
    z = lax.dot_general(aggs, wl_ref[...], (((0,), (1,)), ((), ())),
                        preferred_element_type=jnp.float32)
    out = (z + bl_ref[...][None, :]
           + jnp.dot(h_ref[...], wr_ref[...].T,
                     preferred_element_type=jnp.float32))
    out_ref[...] = jnp.maximum(out, 0.0)


def _layer(aggc, inv, h, wl, bl, wr):
    return pl.pallas_call(
        _layer_body,
        out_shape=jax.ShapeDtypeStruct((N, 256), jnp.float32),
    )(aggc, inv, h, wl, bl, wr)


def _head_body(x1_ref, x2_ref, x3_ref, wjk_ref, bjk_ref,
               wc0, bc0, gc0, bec0, wc1, bc1, gc1, bec1,
               wc2, bc2, gc2, bec2, wc3, bc3, gc3, bec3,
               wo_ref, bo_ref, out_ref):
    wjk = wjk_ref[...]
    h = (jnp.dot(x1_ref[...], wjk[:, 0:256].T, preferred_element_type=jnp.float32)
         + jnp.dot(x2_ref[...], wjk[:, 256:512].T, preferred_element_type=jnp.float32)
         + jnp.dot(x3_ref[...], wjk[:, 512:768].T, preferred_element_type=jnp.float32)
         + bjk_ref[...][None, :])
    for (wc, bc, gc, bec) in ((wc0, bc0, gc0, bec0), (wc1, bc1, gc1, bec1),
                              (wc2, bc2, gc2, bec2), (wc3, bc3, gc3, bec3)):
        h = jnp.dot(h, wc[...].T, preferred_element_type=jnp.float32) + bc[...][None, :]
        mu = jnp.mean(h, axis=0, keepdims=True)
        var = jnp.mean((h - mu) ** 2, axis=0, keepdims=True)
        h = gc[...][None, :] * (h - mu) / jnp.sqrt(var + 1e-5) + bec[...][None, :]
        h = jnp.maximum(h, 0.0)
    o = jnp.sum(h * wo_ref[...], axis=1, keepdims=True) + bo_ref[...]
    out_ref[...] = jax.nn.sigmoid(o)


def _head(x1, x2, x3, wjk, bjk, clf, wo, bo):
    flat = []
    for t in clf:
        flat.extend(t)
    return pl.pallas_call(
        _head_body,
        out_shape=jax.ShapeDtypeStruct((N, 1), jnp.float32),
    )(x1, x2, x3, wjk, bjk, *flat, wo, bo)


def kernel(x, edge_index, W_enc, b_enc, g_enc, beta_enc,
           Wl0, bl0, Wr0, Wl1, bl1, Wr1, Wl2, bl2, Wr2,
           W_jk, b_jk,
           Wc0, bc0, gc0, betac0, Wc1, bc1, gc1, betac1,
           Wc2, bc2, gc2, betac2, Wc3, bc3, gc3, betac3,
           W_out, b_out):
    ei = edge_index.astype(jnp.int32)
    dst = ei[1]
    h, packed = _encoder(x, W_enc, b_enc, g_enc, beta_enc, ei)
    packed = packed.reshape(E)

    zeros1 = jnp.zeros((1, N), jnp.float32)
    zeros4 = jnp.zeros((CPB, N), jnp.float32)

    cnt_parts = _count_kernel(dst, zeros1)
    inv = _prep(cnt_parts.reshape(32, N))

    xs = []
    for (wl, bl, wr) in ((Wl0, bl0, Wr0), (Wl1, bl1, Wr1), (Wl2, bl2, Wr2)):
        hc = _transpose(h).reshape(64, CPB, N)
        aggc = _aggc_kernel(hc, packed, zeros4).reshape(256, N)
        h = _layer(aggc, inv, h, wl, bl, wr)
        xs.append(h)

    clf = ((Wc0, bc0, gc0, betac0), (Wc1, bc1, gc1, betac1),
           (Wc2, bc2, gc2, betac2), (Wc3, bc3, gc3, betac3))
    return _head(xs[0], xs[1], xs[2], W_jk, b_jk, clf, W_out,
                 b_out.reshape(1, 1))
